# baseline (speedup 1.0000x reference)
import functools

import jax
import jax.numpy as jnp
from jax import lax
from jax.experimental import pallas as pl
from jax.experimental.pallas import tpu as pltpu
from jax.experimental.pallas import tpu_sc as plsc

_FIRE = 16
_DEPTH = 1
_GROUP = 128


@functools.lru_cache(maxsize=None)
def _make_gather(batch: int, n_rows: int, d_model: int):
    info = plsc.get_sparse_core_info()
    num_cores, num_subcores = info.num_cores, info.num_subcores
    nl = info.num_lanes
    nw = num_cores * num_subcores
    b_per_w = batch // nw
    n_chunks = b_per_w // _FIRE
    chunks_per_group = _GROUP // _FIRE
    mesh = plsc.VectorSubcoreMesh(core_axis_name="c", subcore_axis_name="s")

    @functools.partial(
        pl.kernel,
        mesh=mesh,
        out_type=jax.ShapeDtypeStruct((d_model, batch), jnp.float32),
        scratch_types=[
            pltpu.VMEM((b_per_w,), jnp.int32),
            pltpu.VMEM((_DEPTH * _FIRE, d_model, 128), jnp.float32),
            pltpu.VMEM((2, d_model, _GROUP), jnp.float32),
            pltpu.SemaphoreType.DMA,
            pltpu.SemaphoreType.DMA,
        ],
        compiler_params=pltpu.CompilerParams(
            use_tc_tiling_on_sc=True, needs_layout_passes=False
        ),
    )
    def gather_kernel(
        idx_hbm, tablet_hbm, out_hbm, idx_v, ring_v, grp_v, sem, sem_out
    ):
        wid = lax.axis_index("s") * num_cores + lax.axis_index("c")
        base = wid * b_per_w
        pltpu.sync_copy(idx_hbm.at[pl.ds(base, b_per_w)], idx_v)

        dvec0 = lax.iota(jnp.int32, nl)
        dvec1 = dvec0 + nl

        def col_dma(i, b):
            j = pl.multiple_of((i // 128) * 128, 128)
            return pltpu.make_async_copy(
                tablet_hbm.at[:, pl.ds(j, 128)],
                ring_v.at[b],
                sem,
            )

        def consume(c, b, v_cur):
            s = (c % _DEPTH) * _FIRE + b
            col_dma(v_cur[b], s).wait()
            l = v_cur[b] % 128
            g = c // chunks_per_group
            lvec = jnp.full((nl,), l, jnp.int32)
            kvec = jnp.full((nl,), (c % chunks_per_group) * _FIRE + b, jnp.int32)
            bvec = jnp.full((nl,), s, jnp.int32)
            gvec = jnp.full((nl,), g % 2, jnp.int32)
            lo = plsc.load_gather(ring_v, [bvec, dvec0, lvec])
            hi = plsc.load_gather(ring_v, [bvec, dvec1, lvec])
            plsc.store_scatter(grp_v, [gvec, dvec0, kvec], lo)
            plsc.store_scatter(grp_v, [gvec, dvec1, kvec], hi)

        def out_copy(g):
            return pltpu.make_async_copy(
                grp_v.at[g % 2],
                out_hbm.at[:, pl.ds(pl.multiple_of(base + g * _GROUP, 128), _GROUP)],
                sem_out,
            )

        def flush(c):
            out_copy(c // chunks_per_group).start()

        for c0 in range(_DEPTH):
            vp = idx_v[pl.ds(c0 * _FIRE, _FIRE)]
            for b in range(_FIRE):
                col_dma(vp[b], c0 * _FIRE + b).start()

        def reuse_wait(c):
            @pl.when((c % chunks_per_group == 0) & (c >= 2 * chunks_per_group))
            def _():
                out_copy(c // chunks_per_group - 2).wait()

        def chunk(c):
            reuse_wait(c)
            v_cur = idx_v[pl.ds(c * _FIRE, _FIRE)]
            v_nxt = idx_v[pl.ds((c + _DEPTH) * _FIRE, _FIRE)]
            for b in range(_FIRE):
                consume(c, b, v_cur)
                col_dma(v_nxt[b], (c % _DEPTH) * _FIRE + b).start()

            @pl.when(c % chunks_per_group == chunks_per_group - 1)
            def _():
                flush(c)

        pl.loop(0, n_chunks - _DEPTH)(chunk)

        def chunk_drain(c):
            reuse_wait(c)
            v_cur = idx_v[pl.ds(c * _FIRE, _FIRE)]
            for b in range(_FIRE):
                consume(c, b, v_cur)

            @pl.when(c % chunks_per_group == chunks_per_group - 1)
            def _():
                flush(c)

        pl.loop(n_chunks - _DEPTH, n_chunks)(chunk_drain)

        n_groups = b_per_w // _GROUP
        out_copy(n_groups - 2).wait()
        out_copy(n_groups - 1).wait()

    return gather_kernel


def kernel(x, table):
    batch = x.shape[0]
    n_rows, d_model = table.shape
    idx = x.reshape(batch).astype(jnp.int32)
    out_t = _make_gather(batch, n_rows, d_model)(idx, table.T)
    return out_t.T.reshape(batch, 1, d_model)
